# Initial kernel scaffold; baseline (speedup 1.0000x reference)
import jax
import jax.numpy as jnp
from jax.experimental import pallas as pl


def kernel(x, Wg, w1, w2, w3, sw1, sw2, sw3):
    raise NotImplementedError("write your pallas kernel here")


# speedup vs baseline: 3.3152x; 3.3152x over previous
import functools

import jax
import jax.numpy as jnp
from jax import lax
from jax.experimental import pallas as pl
from jax.experimental.pallas import tpu as pltpu
from jax.experimental.pallas import tpu_sc as plsc

DIM = 1024
INTER = 512
E = 64
TOP_K = 2
BLK = 128
NTOK = 8192
LPAD = NTOK * TOP_K + E * BLK
NB = LPAD // BLK
GATE_BLK = 512
F32_MIN = float(jnp.finfo(jnp.float32).min)


def _gate_body(x_ref, wg_ref, idx_ref, wgt_ref):
    x = x_ref[...]
    logits = lax.dot_general(x, wg_ref[...], (((1,), (1,)), ((), ())),
                             preferred_element_type=jnp.float32)
    m = jnp.max(logits, axis=-1, keepdims=True)
    ex = jnp.exp(logits - m)
    scores = ex / jnp.sum(ex, axis=-1, keepdims=True)
    cols = lax.broadcasted_iota(jnp.int32, scores.shape, 1)
    m1 = jnp.max(scores, axis=-1, keepdims=True)
    a1 = jnp.min(jnp.where(scores == m1, cols, E), axis=-1, keepdims=True)
    s2 = jnp.where(cols == a1, F32_MIN, scores)
    m2 = jnp.max(s2, axis=-1, keepdims=True)
    a2 = jnp.min(jnp.where(s2 == m2, cols, E), axis=-1, keepdims=True)
    idx_ref[...] = jnp.concatenate([a1, a2], axis=1)
    wgt_ref[...] = jnp.concatenate([m1, m2], axis=1)


def _gate(x2, Wg):
    n = x2.shape[0]
    return pl.pallas_call(
        _gate_body,
        grid=(n // GATE_BLK,),
        in_specs=[
            pl.BlockSpec((GATE_BLK, DIM), lambda i: (i, 0)),
            pl.BlockSpec((E, DIM), lambda i: (0, 0)),
        ],
        out_specs=[
            pl.BlockSpec((GATE_BLK, TOP_K), lambda i: (i, 0)),
            pl.BlockSpec((GATE_BLK, TOP_K), lambda i: (i, 0)),
        ],
        out_shape=[
            jax.ShapeDtypeStruct((n, TOP_K), jnp.int32),
            jax.ShapeDtypeStruct((n, TOP_K), jnp.float32),
        ],
    )(x2, Wg)


def _sc_gather(table, idx):
    n = idx.shape[0]
    d = table.shape[1]
    info = plsc.get_sparse_core_info()
    nw = info.num_cores * info.num_subcores
    per_w = n // nw
    ch = min(per_w, 64)
    steps = per_w // ch
    mesh = plsc.VectorSubcoreMesh(core_axis_name="c", subcore_axis_name="s")

    @functools.partial(
        pl.kernel,
        mesh=mesh,
        out_type=jax.ShapeDtypeStruct((n, d), table.dtype),
        scratch_types=[
            pltpu.VMEM((ch,), jnp.int32),
            pltpu.VMEM((ch, d), table.dtype),
            pltpu.SemaphoreType.DMA,
        ],
    )
    def gather_k(table_hbm, idx_hbm, out_hbm, idx_v, rows_v, sem):
        wid = lax.axis_index("s") * info.num_cores + lax.axis_index("c")
        base = wid * per_w

        def body(i, carry):
            off = base + i * ch
            pltpu.sync_copy(idx_hbm.at[pl.ds(off, ch)], idx_v)
            pltpu.async_copy(table_hbm.at[idx_v], rows_v, sem).wait()
            pltpu.sync_copy(rows_v, out_hbm.at[pl.ds(off, ch)])
            return carry

        lax.fori_loop(0, steps, body, 0)

    return gather_k(table, idx)


def _group_body(eid_ref, xs_ref, w1_ref, w3_ref, w2_ref, sc_ref, ys_ref):
    x = xs_ref[...]
    h1 = lax.dot_general(x, w1_ref[0], (((1,), (1,)), ((), ())),
                         preferred_element_type=jnp.float32)
    h3 = lax.dot_general(x, w3_ref[0], (((1,), (1,)), ((), ())),
                         preferred_element_type=jnp.float32)
    h = (h1 * jax.nn.sigmoid(h1)) * h3
    h = h * sc_ref[0]
    ys_ref[...] = lax.dot_general(h, w2_ref[0], (((1,), (1,)), ((), ())),
                                  preferred_element_type=jnp.float32)


def _grouped_mlp(xs, w1, w2, w3, scale, blk_eid):
    grid_spec = pltpu.PrefetchScalarGridSpec(
        num_scalar_prefetch=1,
        grid=(NB,),
        in_specs=[
            pl.BlockSpec((BLK, DIM), lambda i, e: (i, 0)),
            pl.BlockSpec((1, INTER, DIM), lambda i, e: (e[i], 0, 0)),
            pl.BlockSpec((1, INTER, DIM), lambda i, e: (e[i], 0, 0)),
            pl.BlockSpec((1, DIM, INTER), lambda i, e: (e[i], 0, 0)),
            pl.BlockSpec((1, BLK, 1), lambda i, e: (i, 0, 0)),
        ],
        out_specs=pl.BlockSpec((BLK, DIM), lambda i, e: (i, 0)),
    )
    return pl.pallas_call(
        _group_body,
        grid_spec=grid_spec,
        out_shape=jax.ShapeDtypeStruct((LPAD, DIM), jnp.float32),
    )(blk_eid, xs, w1, w3, w2, scale)


def _final_body(x_ref, sw1_ref, sw3_ref, sw2_ref, g_ref, out_ref):
    x = x_ref[...]
    h1 = lax.dot_general(x, sw1_ref[...], (((1,), (1,)), ((), ())),
                         preferred_element_type=jnp.float32)
    h3 = lax.dot_general(x, sw3_ref[...], (((1,), (1,)), ((), ())),
                         preferred_element_type=jnp.float32)
    h = (h1 * jax.nn.sigmoid(h1)) * h3
    z = lax.dot_general(h, sw2_ref[...], (((1,), (1,)), ((), ())),
                        preferred_element_type=jnp.float32)
    out_ref[...] = z + g_ref[:, 0, :] + g_ref[:, 1, :]


def _final(x2, sw1, sw2, sw3, garr):
    n = x2.shape[0]
    return pl.pallas_call(
        _final_body,
        grid=(n // GATE_BLK,),
        in_specs=[
            pl.BlockSpec((GATE_BLK, DIM), lambda i: (i, 0)),
            pl.BlockSpec((INTER, DIM), lambda i: (0, 0)),
            pl.BlockSpec((INTER, DIM), lambda i: (0, 0)),
            pl.BlockSpec((DIM, INTER), lambda i: (0, 0)),
            pl.BlockSpec((GATE_BLK, TOP_K, DIM), lambda i: (i, 0, 0)),
        ],
        out_specs=pl.BlockSpec((GATE_BLK, DIM), lambda i: (i, 0)),
        out_shape=jax.ShapeDtypeStruct((n, DIM), jnp.float32),
    )(x2, sw1, sw3, sw2, garr)


def kernel(x, Wg, w1, w2, w3, sw1, sw2, sw3):
    shape = x.shape
    x2 = x.reshape(-1, DIM)

    idx, wgt = _gate(x2, Wg)

    eid = idx.reshape(-1).astype(jnp.int32)
    wfl = wgt.reshape(-1)
    na = eid.shape[0]
    order = jnp.argsort(eid, stable=True).astype(jnp.int32)
    sorted_eid = jnp.take(eid, order)
    counts = jnp.bincount(eid, length=E).astype(jnp.int32)
    pc = ((counts + BLK - 1) // BLK) * BLK
    pad_cum = jnp.cumsum(pc).astype(jnp.int32)
    pstart = pad_cum - pc
    cnt_cum = jnp.cumsum(counts).astype(jnp.int32)
    cstart = cnt_cum - counts
    ar = jnp.arange(na, dtype=jnp.int32)
    slot_sorted = (jnp.take(pstart, sorted_eid) + ar
                   - jnp.take(cstart, sorted_eid)).astype(jnp.int32)
    src_tid = jnp.zeros((LPAD,), jnp.int32).at[slot_sorted].set(
        order // TOP_K)
    slot_scale = jnp.zeros((LPAD,), jnp.float32).at[slot_sorted].set(
        jnp.take(wfl, order))
    rpos = jnp.zeros((na,), jnp.int32).at[order].set(slot_sorted)
    blk_eid = jnp.minimum(
        jnp.searchsorted(pad_cum, ar[:NB] * BLK, side="right"),
        E - 1).astype(jnp.int32)

    xs = _sc_gather(x2, src_tid)
    ys = _grouped_mlp(xs, w1, w2, w3, slot_scale.reshape(NB, BLK, 1), blk_eid)
    garr = _sc_gather(ys, rpos).reshape(NTOK, TOP_K, DIM)
    out = _final(x2, sw1, sw2, sw3, garr)
    return out.reshape(shape)


# speedup vs baseline: 4.4898x; 1.3543x over previous
import functools

import jax
import jax.numpy as jnp
from jax import lax
from jax.experimental import pallas as pl
from jax.experimental.pallas import tpu as pltpu
from jax.experimental.pallas import tpu_sc as plsc

DIM = 1024
INTER = 512
E = 64
TOP_K = 2
BLK = 128
NTOK = 8192
LPAD = NTOK * TOP_K + E * BLK
NB = LPAD // BLK
GATE_BLK = 512
F32_MIN = float(jnp.finfo(jnp.float32).min)


def _gate_body(x_ref, wg_ref, idx_ref, wgt_ref):
    x = x_ref[...]
    logits = lax.dot_general(x, wg_ref[...], (((1,), (1,)), ((), ())),
                             preferred_element_type=jnp.float32)
    m = jnp.max(logits, axis=-1, keepdims=True)
    ex = jnp.exp(logits - m)
    scores = ex / jnp.sum(ex, axis=-1, keepdims=True)
    cols = lax.broadcasted_iota(jnp.int32, scores.shape, 1)
    m1 = jnp.max(scores, axis=-1, keepdims=True)
    a1 = jnp.min(jnp.where(scores == m1, cols, E), axis=-1, keepdims=True)
    s2 = jnp.where(cols == a1, F32_MIN, scores)
    m2 = jnp.max(s2, axis=-1, keepdims=True)
    a2 = jnp.min(jnp.where(s2 == m2, cols, E), axis=-1, keepdims=True)
    idx_ref[...] = jnp.concatenate([a1, a2], axis=1)
    wgt_ref[...] = jnp.concatenate([m1, m2], axis=1)


def _gate(x2, Wg):
    n = x2.shape[0]
    return pl.pallas_call(
        _gate_body,
        grid=(n // GATE_BLK,),
        in_specs=[
            pl.BlockSpec((GATE_BLK, DIM), lambda i: (i, 0)),
            pl.BlockSpec((E, DIM), lambda i: (0, 0)),
        ],
        out_specs=[
            pl.BlockSpec((GATE_BLK, TOP_K), lambda i: (i, 0)),
            pl.BlockSpec((GATE_BLK, TOP_K), lambda i: (i, 0)),
        ],
        out_shape=[
            jax.ShapeDtypeStruct((n, TOP_K), jnp.int32),
            jax.ShapeDtypeStruct((n, TOP_K), jnp.float32),
        ],
    )(x2, Wg)


def _sc_gather(table, idx):
    n = idx.shape[0]
    d = table.shape[1]
    info = plsc.get_sparse_core_info()
    nw = info.num_cores * info.num_subcores
    per_w = n // nw
    ch = min(per_w, 64)
    steps = per_w // ch
    mesh = plsc.VectorSubcoreMesh(core_axis_name="c", subcore_axis_name="s")

    @functools.partial(
        pl.kernel,
        mesh=mesh,
        out_type=jax.ShapeDtypeStruct((n, d), table.dtype),
        scratch_types=[
            pltpu.VMEM((ch,), jnp.int32),
            pltpu.VMEM((ch, d), table.dtype),
            pltpu.SemaphoreType.DMA,
        ],
    )
    def gather_k(table_hbm, idx_hbm, out_hbm, idx_v, rows_v, sem):
        wid = lax.axis_index("s") * info.num_cores + lax.axis_index("c")
        base = wid * per_w

        def body(i, carry):
            off = base + i * ch
            pltpu.sync_copy(idx_hbm.at[pl.ds(off, ch)], idx_v)
            pltpu.async_copy(table_hbm.at[idx_v], rows_v, sem).wait()
            pltpu.sync_copy(rows_v, out_hbm.at[pl.ds(off, ch)])
            return carry

        lax.fori_loop(0, steps, body, 0)

    return gather_k(table, idx)


def _group_body(eid_ref, xs_ref, w1_ref, w3_ref, w2_ref, sc_ref, ys_ref):
    x = xs_ref[...]
    h1 = lax.dot_general(x, w1_ref[0], (((1,), (1,)), ((), ())),
                         preferred_element_type=jnp.float32)
    h3 = lax.dot_general(x, w3_ref[0], (((1,), (1,)), ((), ())),
                         preferred_element_type=jnp.float32)
    h = (h1 * jax.nn.sigmoid(h1)) * h3
    h = h * sc_ref[0]
    ys_ref[...] = lax.dot_general(h, w2_ref[0], (((1,), (1,)), ((), ())),
                                  preferred_element_type=jnp.float32)


def _grouped_mlp(xs, w1, w2, w3, scale, blk_eid):
    grid_spec = pltpu.PrefetchScalarGridSpec(
        num_scalar_prefetch=1,
        grid=(NB,),
        in_specs=[
            pl.BlockSpec((BLK, DIM), lambda i, e: (i, 0)),
            pl.BlockSpec((1, INTER, DIM), lambda i, e: (e[i], 0, 0)),
            pl.BlockSpec((1, INTER, DIM), lambda i, e: (e[i], 0, 0)),
            pl.BlockSpec((1, DIM, INTER), lambda i, e: (e[i], 0, 0)),
            pl.BlockSpec((1, BLK, 1), lambda i, e: (i, 0, 0)),
        ],
        out_specs=pl.BlockSpec((BLK, DIM), lambda i, e: (i, 0)),
    )
    return pl.pallas_call(
        _group_body,
        grid_spec=grid_spec,
        out_shape=jax.ShapeDtypeStruct((LPAD, DIM), jnp.float32),
    )(blk_eid, xs, w1, w3, w2, scale)


def _final_body(x_ref, sw1_ref, sw3_ref, sw2_ref, g_ref, out_ref):
    x = x_ref[...]
    h1 = lax.dot_general(x, sw1_ref[...], (((1,), (1,)), ((), ())),
                         preferred_element_type=jnp.float32)
    h3 = lax.dot_general(x, sw3_ref[...], (((1,), (1,)), ((), ())),
                         preferred_element_type=jnp.float32)
    h = (h1 * jax.nn.sigmoid(h1)) * h3
    z = lax.dot_general(h, sw2_ref[...], (((1,), (1,)), ((), ())),
                        preferred_element_type=jnp.float32)
    out_ref[...] = z + g_ref[:, 0, :] + g_ref[:, 1, :]


def _final(x2, sw1, sw2, sw3, garr):
    n = x2.shape[0]
    return pl.pallas_call(
        _final_body,
        grid=(n // GATE_BLK,),
        in_specs=[
            pl.BlockSpec((GATE_BLK, DIM), lambda i: (i, 0)),
            pl.BlockSpec((INTER, DIM), lambda i: (0, 0)),
            pl.BlockSpec((INTER, DIM), lambda i: (0, 0)),
            pl.BlockSpec((DIM, INTER), lambda i: (0, 0)),
            pl.BlockSpec((GATE_BLK, TOP_K, DIM), lambda i: (i, 0, 0)),
        ],
        out_specs=pl.BlockSpec((GATE_BLK, DIM), lambda i: (i, 0)),
        out_shape=jax.ShapeDtypeStruct((n, DIM), jnp.float32),
    )(x2, sw1, sw3, sw2, garr)


def kernel(x, Wg, w1, w2, w3, sw1, sw2, sw3):
    shape = x.shape
    x2 = x.reshape(-1, DIM)

    idx, wgt = _gate(x2, Wg)

    eid = idx.reshape(-1).astype(jnp.int32)
    wfl = wgt.reshape(-1)
    na = eid.shape[0]
    order = jnp.argsort(eid, stable=True).astype(jnp.int32)
    sorted_eid = jnp.take(eid, order)
    counts = jnp.bincount(eid, length=E).astype(jnp.int32)
    pc = ((counts + BLK - 1) // BLK) * BLK
    pad_cum = jnp.cumsum(pc).astype(jnp.int32)
    pstart = pad_cum - pc
    cnt_cum = jnp.cumsum(counts).astype(jnp.int32)
    cstart = cnt_cum - counts
    ar = jnp.arange(na, dtype=jnp.int32)
    slot_sorted = (jnp.take(pstart, sorted_eid) + ar
                   - jnp.take(cstart, sorted_eid)).astype(jnp.int32)
    src_tid = (jnp.arange(LPAD, dtype=jnp.int32) % NTOK).at[slot_sorted].set(
        order // TOP_K)
    slot_scale = jnp.zeros((LPAD,), jnp.float32).at[slot_sorted].set(
        jnp.take(wfl, order))
    rpos = jnp.zeros((na,), jnp.int32).at[order].set(slot_sorted)
    blk_eid = jnp.minimum(
        jnp.searchsorted(pad_cum, ar[:NB] * BLK, side="right"),
        E - 1).astype(jnp.int32)

    xs = _sc_gather(x2, src_tid)
    ys = _grouped_mlp(xs, w1, w2, w3, slot_scale.reshape(NB, BLK, 1), blk_eid)
    garr = _sc_gather(ys, rpos).reshape(NTOK, TOP_K, DIM)
    out = _final(x2, sw1, sw2, sw3, garr)
    return out.reshape(shape)
